# baseline (speedup 1.0000x reference)
import jax
import jax.numpy as jnp
from jax import lax
from jax.experimental import pallas as pl
from jax.experimental.pallas import tpu as pltpu

_N = 87296
_C = 91
_R = 341
_L = 256
_BN = 2816
_G = 31
_IOU = 0.6
_K = 100


def _prep_kernel(logits_ref, ctr_ref, anc_ref, reg_ref, pack_ref):
    lg = logits_ref[...]
    ct = ctr_ref[...]
    s = jnp.sqrt(jax.nn.sigmoid(lg) * jax.nn.sigmoid(ct))
    mx = jnp.max(s, axis=1, keepdims=True)
    lane = lax.broadcasted_iota(jnp.int32, s.shape, 1)
    lbl = jnp.min(jnp.where(s == mx, lane, _C), axis=1, keepdims=True)
    cols = jnp.concatenate(
        [anc_ref[...], reg_ref[...], mx, lbl.astype(jnp.float32)], axis=1)
    pack_ref[...] = cols.T


def _nms_kernel(pack_ref, out_ref, x1s, y1s, x2s, y2s, ars, scs):
    ax1 = pack_ref[0]
    ay1 = pack_ref[1]
    ax2 = pack_ref[2]
    ay2 = pack_ref[3]
    cx = 0.5 * (ax1 + ax2)
    cy = 0.5 * (ay1 + ay2)
    w = ax2 - ax1
    h = ay2 - ay1
    x1 = cx - pack_ref[4] * w
    y1 = cy - pack_ref[5] * h
    x2 = cx + pack_ref[6] * w
    y2 = cy + pack_ref[7] * h
    x1s[...] = x1
    y1s[...] = y1
    x2s[...] = x2
    y2s[...] = y2
    ars[...] = jnp.maximum(x2 - x1, 0.0) * jnp.maximum(y2 - y1, 0.0)
    scs[...] = pack_ref[8]

    neginf = jnp.float32(-jnp.inf)

    def body(i, carry):
        s = scs[...]
        m = jnp.max(jnp.max(s, axis=0, keepdims=True), axis=1, keepdims=True)
        flat = (lax.broadcasted_iota(jnp.int32, s.shape, 0) * _L
                + lax.broadcasted_iota(jnp.int32, s.shape, 1))
        idxv = jnp.where(s == m, flat, _N)
        idxv = jnp.min(jnp.min(idxv, axis=0, keepdims=True), axis=1,
                       keepdims=True)
        idx = idxv[0, 0]
        r = lax.shift_right_logical(idx, 8)
        c = lax.bitwise_and(idx, 255)
        lane = lax.broadcasted_iota(jnp.int32, (1, _L), 1)
        sel = lane == c

        def pickf(ref):
            row = ref[pl.ds(r, 1), :]
            return jnp.max(jnp.where(sel, row, neginf), axis=1, keepdims=True)

        bx1 = pickf(x1s)
        by1 = pickf(y1s)
        bx2 = pickf(x2s)
        by2 = pickf(y2s)
        bar = pickf(ars)
        srow = pack_ref[8, pl.ds(r, 1), :]
        bsc = jnp.max(jnp.where(sel, srow, neginf), axis=1, keepdims=True)
        lrow = pack_ref[9, pl.ds(r, 1), :]
        blb = jnp.max(jnp.where(sel, lrow, neginf), axis=1, keepdims=True)

        iw = jnp.maximum(jnp.minimum(x2s[...], bx2)
                         - jnp.maximum(x1s[...], bx1), 0.0)
        ih = jnp.maximum(jnp.minimum(y2s[...], by2)
                         - jnp.maximum(y1s[...], by1), 0.0)
        inter = iw * ih
        iou = inter / (bar + ars[...] - inter)
        scs[...] = jnp.where(iou > _IOU, neginf, s)

        li = lax.broadcasted_iota(jnp.int32, (1, 128), 1)
        v = jnp.where(
            li == 0, bx1,
            jnp.where(li == 1, by1,
                      jnp.where(li == 2, bx2,
                                jnp.where(li == 3, by2,
                                          jnp.where(li == 4, bsc,
                                                    jnp.where(li == 5, blb,
                                                              0.0))))))
        out_ref[pl.ds(i, 1)] = v.reshape(1, 1, 128)
        return carry

    lax.fori_loop(0, _K, body, 0)


def _prep_call(class_logits, box_ctrness, anchors, box_regression):
    return pl.pallas_call(
        _prep_kernel,
        grid=(_G,),
        in_specs=[
            pl.BlockSpec((_BN, _C), lambda i: (i, 0)),
            pl.BlockSpec((_BN, 1), lambda i: (i, 0)),
            pl.BlockSpec((_BN, 4), lambda i: (i, 0)),
            pl.BlockSpec((_BN, 4), lambda i: (i, 0)),
        ],
        out_specs=pl.BlockSpec((10, _BN), lambda i: (0, i)),
        out_shape=jax.ShapeDtypeStruct((10, _N), jnp.float32),
        compiler_params=pltpu.CompilerParams(
            dimension_semantics=("arbitrary",),
        ),
        name="fcos_prep",
    )(class_logits, box_ctrness, anchors, box_regression)


def _nms_call(pack_p):
    return pl.pallas_call(
        _nms_kernel,
        out_shape=jax.ShapeDtypeStruct((_K, 1, 128), jnp.float32),
        scratch_shapes=[pltpu.VMEM((_R, _L), jnp.float32)] * 6,
        name="fcos_nms",
    )(pack_p)


def kernel(class_logits, box_regression, box_ctrness, anchors):
    pack = _prep_call(class_logits, box_ctrness, anchors, box_regression)
    pack_p = pack.reshape(10, _R, _L)
    out = _nms_call(pack_p).reshape(_K, 128)
    pred_boxes = out[:, :4]
    scores = out[:, 4]
    labels = out[:, 5].astype(jnp.int32)
    return pred_boxes, labels, scores


# speedup vs baseline: 2.3143x; 2.3143x over previous
import jax
import jax.numpy as jnp
from jax import lax
from jax.experimental import pallas as pl
from jax.experimental.pallas import tpu as pltpu

_N = 87296
_C = 91
_R = 341
_L = 256
_BN = 2816
_G = 31
_IOU = 0.6
_K = 100


def _prep_kernel(logits_ref, ctr_ref, anc_ref, reg_ref, pack_ref):
    lg = logits_ref[...]
    ct = ctr_ref[...]
    s = jnp.sqrt(jax.nn.sigmoid(lg) * jax.nn.sigmoid(ct))
    mx = jnp.max(s, axis=0, keepdims=True)
    row = lax.broadcasted_iota(jnp.int32, s.shape, 0)
    lbl = jnp.min(jnp.where(s == mx, row, _C), axis=0, keepdims=True)
    pack_ref[...] = jnp.concatenate(
        [anc_ref[...], reg_ref[...], mx, lbl.astype(jnp.float32)], axis=0)


def _nms_kernel(pack_ref, out_ref, x1s, y1s, x2s, y2s, ars, scs):
    ax1 = pack_ref[0]
    ay1 = pack_ref[1]
    ax2 = pack_ref[2]
    ay2 = pack_ref[3]
    cx = 0.5 * (ax1 + ax2)
    cy = 0.5 * (ay1 + ay2)
    w = ax2 - ax1
    h = ay2 - ay1
    x1 = cx - pack_ref[4] * w
    y1 = cy - pack_ref[5] * h
    x2 = cx + pack_ref[6] * w
    y2 = cy + pack_ref[7] * h
    x1s[...] = x1
    y1s[...] = y1
    x2s[...] = x2
    y2s[...] = y2
    ars[...] = jnp.maximum(x2 - x1, 0.0) * jnp.maximum(y2 - y1, 0.0)
    scs[...] = pack_ref[8]

    neginf = jnp.float32(-jnp.inf)

    def body(i, carry):
        s = scs[...]
        m = jnp.max(jnp.max(s, axis=0, keepdims=True), axis=1, keepdims=True)
        flat = (lax.broadcasted_iota(jnp.int32, s.shape, 0) * _L
                + lax.broadcasted_iota(jnp.int32, s.shape, 1))
        idxv = jnp.where(s == m, flat, _N)
        idxv = jnp.min(jnp.min(idxv, axis=0, keepdims=True), axis=1,
                       keepdims=True)
        idx = idxv[0, 0]
        r = lax.shift_right_logical(idx, 8)
        c = lax.bitwise_and(idx, 255)
        lane = lax.broadcasted_iota(jnp.int32, (1, _L), 1)
        sel = lane == c

        def pickf(ref):
            row = ref[pl.ds(r, 1), :]
            return jnp.max(jnp.where(sel, row, neginf), axis=1, keepdims=True)

        bx1 = pickf(x1s)
        by1 = pickf(y1s)
        bx2 = pickf(x2s)
        by2 = pickf(y2s)
        bar = pickf(ars)
        srow = pack_ref[8, pl.ds(r, 1), :]
        bsc = jnp.max(jnp.where(sel, srow, neginf), axis=1, keepdims=True)
        lrow = pack_ref[9, pl.ds(r, 1), :]
        blb = jnp.max(jnp.where(sel, lrow, neginf), axis=1, keepdims=True)

        iw = jnp.maximum(jnp.minimum(x2s[...], bx2)
                         - jnp.maximum(x1s[...], bx1), 0.0)
        ih = jnp.maximum(jnp.minimum(y2s[...], by2)
                         - jnp.maximum(y1s[...], by1), 0.0)
        inter = iw * ih
        iou = inter / (bar + ars[...] - inter)
        scs[...] = jnp.where(iou > _IOU, neginf, s)

        li = lax.broadcasted_iota(jnp.int32, (1, 128), 1)
        v = jnp.where(
            li == 0, bx1,
            jnp.where(li == 1, by1,
                      jnp.where(li == 2, bx2,
                                jnp.where(li == 3, by2,
                                          jnp.where(li == 4, bsc,
                                                    jnp.where(li == 5, blb,
                                                              0.0))))))
        out_ref[pl.ds(i, 1)] = v.reshape(1, 1, 128)
        return carry

    lax.fori_loop(0, _K, body, 0)


def _prep_call(class_logits, box_ctrness, anchors, box_regression):
    return pl.pallas_call(
        _prep_kernel,
        grid=(_G,),
        in_specs=[
            pl.BlockSpec((_C, _BN), lambda i: (0, i)),
            pl.BlockSpec((1, _BN), lambda i: (0, i)),
            pl.BlockSpec((4, _BN), lambda i: (0, i)),
            pl.BlockSpec((4, _BN), lambda i: (0, i)),
        ],
        out_specs=pl.BlockSpec((10, _BN), lambda i: (0, i)),
        out_shape=jax.ShapeDtypeStruct((10, _N), jnp.float32),
        compiler_params=pltpu.CompilerParams(
            dimension_semantics=("arbitrary",),
        ),
        name="fcos_prep",
    )(class_logits, box_ctrness, anchors, box_regression)


def _nms_call(pack_p):
    return pl.pallas_call(
        _nms_kernel,
        out_shape=jax.ShapeDtypeStruct((_K, 1, 128), jnp.float32),
        scratch_shapes=[pltpu.VMEM((_R, _L), jnp.float32)] * 6,
        name="fcos_nms",
    )(pack_p)


def kernel(class_logits, box_regression, box_ctrness, anchors):
    pack = _prep_call(class_logits.T, box_ctrness.T,
                      anchors.T, box_regression.T)
    pack_p = pack.reshape(10, _R, _L)
    out = _nms_call(pack_p).reshape(_K, 128)
    pred_boxes = out[:, :4]
    scores = out[:, 4]
    labels = out[:, 5].astype(jnp.int32)
    return pred_boxes, labels, scores


# speedup vs baseline: 2.3543x; 1.0173x over previous
import jax
import jax.numpy as jnp
from jax import lax
from jax.experimental import pallas as pl
from jax.experimental.pallas import tpu as pltpu

_N = 87296
_C = 91
_R = 341
_L = 256
_BN = 2816
_G = 31
_IOU = 0.6
_K = 100


def _prep_kernel(logits_ref, ctr_ref, anc_ref, reg_ref, pack_ref):
    lg = logits_ref[...]
    ct = ctr_ref[...]
    s = jnp.sqrt(jax.nn.sigmoid(lg) * jax.nn.sigmoid(ct))
    mx = jnp.max(s, axis=0, keepdims=True)
    row = lax.broadcasted_iota(jnp.int32, s.shape, 0)
    lbl = jnp.min(jnp.where(s == mx, row, _C), axis=0, keepdims=True)
    pack_ref[...] = jnp.concatenate(
        [anc_ref[...], reg_ref[...], mx, lbl.astype(jnp.float32)], axis=0)


def _nms_kernel(pack_ref, out_ref, x1s, y1s, x2s, y2s, ars, scs):
    ax1 = pack_ref[0]
    ay1 = pack_ref[1]
    ax2 = pack_ref[2]
    ay2 = pack_ref[3]
    cx = 0.5 * (ax1 + ax2)
    cy = 0.5 * (ay1 + ay2)
    w = ax2 - ax1
    h = ay2 - ay1
    x1 = cx - pack_ref[4] * w
    y1 = cy - pack_ref[5] * h
    x2 = cx + pack_ref[6] * w
    y2 = cy + pack_ref[7] * h
    x1s[...] = x1
    y1s[...] = y1
    x2s[...] = x2
    y2s[...] = y2
    ars[...] = jnp.maximum(x2 - x1, 0.0) * jnp.maximum(y2 - y1, 0.0)
    s0 = pack_ref[8]
    scs[...] = s0

    neginf = jnp.float32(-jnp.inf)
    rowio = lax.broadcasted_iota(jnp.int32, (_R, _L), 0)
    colmax0 = jnp.max(s0, axis=0, keepdims=True)
    colrow0 = jnp.min(jnp.where(s0 == colmax0, rowio, _R), axis=0,
                      keepdims=True)

    def body(i, carry):
        colmax, colrow = carry
        lane = lax.broadcasted_iota(jnp.int32, (1, _L), 1)
        m = jnp.max(colmax, axis=1, keepdims=True)
        cand = jnp.where(colmax == m, colrow * _L + lane, _N)
        idx = jnp.min(cand, axis=1, keepdims=True)[0, 0]
        r = lax.shift_right_logical(idx, 8)
        c = lax.bitwise_and(idx, 255)
        sel = lane == c

        def pickf(ref):
            row = ref[pl.ds(r, 1), :]
            return jnp.max(jnp.where(sel, row, neginf), axis=1, keepdims=True)

        bx1 = pickf(x1s)
        by1 = pickf(y1s)
        bx2 = pickf(x2s)
        by2 = pickf(y2s)
        bar = pickf(ars)
        srow = pack_ref[8, pl.ds(r, 1), :]
        bsc = jnp.max(jnp.where(sel, srow, neginf), axis=1, keepdims=True)
        lrow = pack_ref[9, pl.ds(r, 1), :]
        blb = jnp.max(jnp.where(sel, lrow, neginf), axis=1, keepdims=True)

        s = scs[...]
        iw = jnp.maximum(jnp.minimum(x2s[...], bx2)
                         - jnp.maximum(x1s[...], bx1), 0.0)
        ih = jnp.maximum(jnp.minimum(y2s[...], by2)
                         - jnp.maximum(y1s[...], by1), 0.0)
        inter = iw * ih
        iou = inter / (bar + ars[...] - inter)
        new_s = jnp.where(iou > _IOU, neginf, s)
        scs[...] = new_s
        colmax2 = jnp.max(new_s, axis=0, keepdims=True)
        colrow2 = jnp.min(jnp.where(new_s == colmax2, rowio, _R), axis=0,
                          keepdims=True)

        li = lax.broadcasted_iota(jnp.int32, (1, 128), 1)
        v = jnp.where(
            li == 0, bx1,
            jnp.where(li == 1, by1,
                      jnp.where(li == 2, bx2,
                                jnp.where(li == 3, by2,
                                          jnp.where(li == 4, bsc,
                                                    jnp.where(li == 5, blb,
                                                              0.0))))))
        out_ref[pl.ds(i, 1)] = v.reshape(1, 1, 128)
        return (colmax2, colrow2)

    lax.fori_loop(0, _K, body, (colmax0, colrow0))


def _prep_call(class_logits, box_ctrness, anchors, box_regression):
    return pl.pallas_call(
        _prep_kernel,
        grid=(_G,),
        in_specs=[
            pl.BlockSpec((_C, _BN), lambda i: (0, i)),
            pl.BlockSpec((1, _BN), lambda i: (0, i)),
            pl.BlockSpec((4, _BN), lambda i: (0, i)),
            pl.BlockSpec((4, _BN), lambda i: (0, i)),
        ],
        out_specs=pl.BlockSpec((10, _BN), lambda i: (0, i)),
        out_shape=jax.ShapeDtypeStruct((10, _N), jnp.float32),
        compiler_params=pltpu.CompilerParams(
            dimension_semantics=("arbitrary",),
        ),
        name="fcos_prep",
    )(class_logits, box_ctrness, anchors, box_regression)


def _nms_call(pack_p):
    return pl.pallas_call(
        _nms_kernel,
        out_shape=jax.ShapeDtypeStruct((_K, 1, 128), jnp.float32),
        scratch_shapes=[pltpu.VMEM((_R, _L), jnp.float32)] * 6,
        name="fcos_nms",
    )(pack_p)


def kernel(class_logits, box_regression, box_ctrness, anchors):
    pack = _prep_call(class_logits.T, box_ctrness.T,
                      anchors.T, box_regression.T)
    pack_p = pack.reshape(10, _R, _L)
    out = _nms_call(pack_p).reshape(_K, 128)
    pred_boxes = out[:, :4]
    scores = out[:, 4]
    labels = out[:, 5].astype(jnp.int32)
    return pred_boxes, labels, scores


# speedup vs baseline: 2.3932x; 1.0165x over previous
import jax
import jax.numpy as jnp
from jax import lax
from jax.experimental import pallas as pl
from jax.experimental.pallas import tpu as pltpu

_N = 87296
_C = 91
_R = 341
_L = 256
_BN = 2816
_G = 31
_IOU = 0.6
_K = 100


def _prep_kernel(logits_ref, ctr_ref, anc_ref, reg_ref, pack_ref):
    lg = logits_ref[...]
    ct = ctr_ref[...]
    s = jnp.sqrt(jax.nn.sigmoid(lg) * jax.nn.sigmoid(ct))
    mx = jnp.max(s, axis=0, keepdims=True)
    row = lax.broadcasted_iota(jnp.int32, s.shape, 0)
    lbl = jnp.min(jnp.where(s == mx, row, _C), axis=0, keepdims=True)
    pack_ref[...] = jnp.concatenate(
        [anc_ref[...], reg_ref[...], mx, lbl.astype(jnp.float32)], axis=0)


def _nms_kernel(pack_ref, out_ref, x1s, y1s, x2s, y2s, ars, scs):
    ax1 = pack_ref[0]
    ay1 = pack_ref[1]
    ax2 = pack_ref[2]
    ay2 = pack_ref[3]
    cx = 0.5 * (ax1 + ax2)
    cy = 0.5 * (ay1 + ay2)
    w = ax2 - ax1
    h = ay2 - ay1
    x1 = cx - pack_ref[4] * w
    y1 = cy - pack_ref[5] * h
    x2 = cx + pack_ref[6] * w
    y2 = cy + pack_ref[7] * h
    x1s[...] = x1
    y1s[...] = y1
    x2s[...] = x2
    y2s[...] = y2
    ars[...] = jnp.maximum(x2 - x1, 0.0) * jnp.maximum(y2 - y1, 0.0)
    s0 = pack_ref[8]
    scs[...] = s0

    neginf = jnp.float32(-jnp.inf)
    rowio = lax.broadcasted_iota(jnp.int32, (_R, _L), 0)
    colmax0 = jnp.max(s0, axis=0, keepdims=True)
    colrow0 = jnp.min(jnp.where(s0 == colmax0, rowio, _R), axis=0,
                      keepdims=True)

    def body(i, carry):
        colmax, colrow = carry
        lane = lax.broadcasted_iota(jnp.int32, (1, _L), 1)
        m = jnp.max(colmax, axis=1, keepdims=True)
        cand = jnp.where(colmax == m, colrow * _L + lane, _N)
        idx = jnp.min(cand, axis=1, keepdims=True)[0, 0]
        r = lax.shift_right_logical(idx, 8)
        c = lax.bitwise_and(idx, 255)
        sel = lane == c

        def pickf(ref):
            row = ref[pl.ds(r, 1), :]
            return jnp.max(jnp.where(sel, row, neginf), axis=1, keepdims=True)

        bx1 = pickf(x1s)
        by1 = pickf(y1s)
        bx2 = pickf(x2s)
        by2 = pickf(y2s)
        bar = pickf(ars)
        srow = pack_ref[8, pl.ds(r, 1), :]
        bsc = jnp.max(jnp.where(sel, srow, neginf), axis=1, keepdims=True)
        lrow = pack_ref[9, pl.ds(r, 1), :]
        blb = jnp.max(jnp.where(sel, lrow, neginf), axis=1, keepdims=True)

        s = scs[...]
        iw = jnp.maximum(jnp.minimum(x2s[...], bx2)
                         - jnp.maximum(x1s[...], bx1), 0.0)
        ih = jnp.maximum(jnp.minimum(y2s[...], by2)
                         - jnp.maximum(y1s[...], by1), 0.0)
        inter = iw * ih
        iou = inter / (bar + ars[...] - inter)
        new_s = jnp.where(iou > _IOU, neginf, s)
        scs[...] = new_s
        colmax2 = jnp.max(new_s, axis=0, keepdims=True)
        colrow2 = jnp.min(jnp.where(new_s == colmax2, rowio, _R), axis=0,
                          keepdims=True)

        li = lax.broadcasted_iota(jnp.int32, (1, 128), 1)
        v = jnp.where(
            li == 0, bx1,
            jnp.where(li == 1, by1,
                      jnp.where(li == 2, bx2,
                                jnp.where(li == 3, by2,
                                          jnp.where(li == 4, bsc,
                                                    jnp.where(li == 5, blb,
                                                              0.0))))))
        out_ref[pl.ds(i, 1)] = v.reshape(1, 1, 128)
        return (colmax2, colrow2)

    lax.fori_loop(0, _K, body, (colmax0, colrow0), unroll=2)


def _prep_call(class_logits, box_ctrness, anchors, box_regression):
    return pl.pallas_call(
        _prep_kernel,
        grid=(_G,),
        in_specs=[
            pl.BlockSpec((_C, _BN), lambda i: (0, i)),
            pl.BlockSpec((1, _BN), lambda i: (0, i)),
            pl.BlockSpec((4, _BN), lambda i: (0, i)),
            pl.BlockSpec((4, _BN), lambda i: (0, i)),
        ],
        out_specs=pl.BlockSpec((10, _BN), lambda i: (0, i)),
        out_shape=jax.ShapeDtypeStruct((10, _N), jnp.float32),
        compiler_params=pltpu.CompilerParams(
            dimension_semantics=("arbitrary",),
        ),
        name="fcos_prep",
    )(class_logits, box_ctrness, anchors, box_regression)


def _nms_call(pack_p):
    return pl.pallas_call(
        _nms_kernel,
        out_shape=jax.ShapeDtypeStruct((_K, 1, 128), jnp.float32),
        scratch_shapes=[pltpu.VMEM((_R, _L), jnp.float32)] * 6,
        name="fcos_nms",
    )(pack_p)


def kernel(class_logits, box_regression, box_ctrness, anchors):
    pack = _prep_call(class_logits.T, box_ctrness.T,
                      anchors.T, box_regression.T)
    pack_p = pack.reshape(10, _R, _L)
    out = _nms_call(pack_p).reshape(_K, 128)
    pred_boxes = out[:, :4]
    scores = out[:, 4]
    labels = out[:, 5].astype(jnp.int32)
    return pred_boxes, labels, scores


# speedup vs baseline: 2.6435x; 1.1046x over previous
import jax
import jax.numpy as jnp
from jax import lax
from jax.experimental import pallas as pl
from jax.experimental.pallas import tpu as pltpu

_N = 87296
_C = 91
_R = 341
_L = 256
_BN = 7936
_G = 11
_IOU = 0.6
_K = 100


def _prep_kernel(logits_ref, ctr_ref, anc_ref, reg_ref, pack_ref):
    lg = logits_ref[...]
    ct = ctr_ref[...]
    s = jnp.sqrt(jax.nn.sigmoid(lg) * jax.nn.sigmoid(ct))
    mx = jnp.max(s, axis=0, keepdims=True)
    row = lax.broadcasted_iota(jnp.int32, s.shape, 0)
    lbl = jnp.min(jnp.where(s == mx, row, _C), axis=0, keepdims=True)
    pack_ref[...] = jnp.concatenate(
        [anc_ref[...], reg_ref[...], mx, lbl.astype(jnp.float32)], axis=0)


def _nms_kernel(pack_ref, out_ref, x1s, y1s, x2s, y2s, ars, scs):
    ax1 = pack_ref[0]
    ay1 = pack_ref[1]
    ax2 = pack_ref[2]
    ay2 = pack_ref[3]
    cx = 0.5 * (ax1 + ax2)
    cy = 0.5 * (ay1 + ay2)
    w = ax2 - ax1
    h = ay2 - ay1
    x1 = cx - pack_ref[4] * w
    y1 = cy - pack_ref[5] * h
    x2 = cx + pack_ref[6] * w
    y2 = cy + pack_ref[7] * h
    x1s[...] = x1
    y1s[...] = y1
    x2s[...] = x2
    y2s[...] = y2
    ars[...] = jnp.maximum(x2 - x1, 0.0) * jnp.maximum(y2 - y1, 0.0)
    s0 = pack_ref[8]
    scs[...] = s0

    neginf = jnp.float32(-jnp.inf)
    rowio = lax.broadcasted_iota(jnp.int32, (_R, _L), 0)
    colmax0 = jnp.max(s0, axis=0, keepdims=True)
    colrow0 = jnp.min(jnp.where(s0 == colmax0, rowio, _R), axis=0,
                      keepdims=True)

    def body(i, carry):
        colmax, colrow = carry
        lane = lax.broadcasted_iota(jnp.int32, (1, _L), 1)
        m = jnp.max(colmax, axis=1, keepdims=True)
        cand = jnp.where(colmax == m, colrow * _L + lane, _N)
        idx = jnp.min(cand, axis=1, keepdims=True)[0, 0]
        r = lax.shift_right_logical(idx, 8)
        c = lax.bitwise_and(idx, 255)
        sel = lane == c

        def pickf(ref):
            row = ref[pl.ds(r, 1), :]
            return jnp.max(jnp.where(sel, row, neginf), axis=1, keepdims=True)

        bx1 = pickf(x1s)
        by1 = pickf(y1s)
        bx2 = pickf(x2s)
        by2 = pickf(y2s)
        bar = pickf(ars)
        srow = pack_ref[8, pl.ds(r, 1), :]
        bsc = jnp.max(jnp.where(sel, srow, neginf), axis=1, keepdims=True)
        lrow = pack_ref[9, pl.ds(r, 1), :]
        blb = jnp.max(jnp.where(sel, lrow, neginf), axis=1, keepdims=True)

        s = scs[...]
        iw = jnp.maximum(jnp.minimum(x2s[...], bx2)
                         - jnp.maximum(x1s[...], bx1), 0.0)
        ih = jnp.maximum(jnp.minimum(y2s[...], by2)
                         - jnp.maximum(y1s[...], by1), 0.0)
        inter = iw * ih
        iou = inter / (bar + ars[...] - inter)
        new_s = jnp.where(iou > _IOU, neginf, s)
        scs[...] = new_s
        colmax2 = jnp.max(new_s, axis=0, keepdims=True)
        colrow2 = jnp.min(jnp.where(new_s == colmax2, rowio, _R), axis=0,
                          keepdims=True)

        li = lax.broadcasted_iota(jnp.int32, (1, 128), 1)
        v = jnp.where(
            li == 0, bx1,
            jnp.where(li == 1, by1,
                      jnp.where(li == 2, bx2,
                                jnp.where(li == 3, by2,
                                          jnp.where(li == 4, bsc,
                                                    jnp.where(li == 5, blb,
                                                              0.0))))))
        out_ref[pl.ds(i, 1)] = v.reshape(1, 1, 128)
        return (colmax2, colrow2)

    lax.fori_loop(0, _K, body, (colmax0, colrow0), unroll=4)


def _prep_call(class_logits, box_ctrness, anchors, box_regression):
    return pl.pallas_call(
        _prep_kernel,
        grid=(_G,),
        in_specs=[
            pl.BlockSpec((_C, _BN), lambda i: (0, i)),
            pl.BlockSpec((1, _BN), lambda i: (0, i)),
            pl.BlockSpec((4, _BN), lambda i: (0, i)),
            pl.BlockSpec((4, _BN), lambda i: (0, i)),
        ],
        out_specs=pl.BlockSpec((10, _BN), lambda i: (0, i)),
        out_shape=jax.ShapeDtypeStruct((10, _N), jnp.float32),
        compiler_params=pltpu.CompilerParams(
            dimension_semantics=("arbitrary",),
        ),
        name="fcos_prep",
    )(class_logits, box_ctrness, anchors, box_regression)


def _nms_call(pack_p):
    return pl.pallas_call(
        _nms_kernel,
        out_shape=jax.ShapeDtypeStruct((_K, 1, 128), jnp.float32),
        scratch_shapes=[pltpu.VMEM((_R, _L), jnp.float32)] * 6,
        name="fcos_nms",
    )(pack_p)


def kernel(class_logits, box_regression, box_ctrness, anchors):
    pack = _prep_call(class_logits.T, box_ctrness.T,
                      anchors.T, box_regression.T)
    pack_p = pack.reshape(10, _R, _L)
    out = _nms_call(pack_p).reshape(_K, 128)
    pred_boxes = out[:, :4]
    scores = out[:, 4]
    labels = out[:, 5].astype(jnp.int32)
    return pred_boxes, labels, scores


# speedup vs baseline: 2.6840x; 1.0153x over previous
import jax
import jax.numpy as jnp
from jax import lax
from jax.experimental import pallas as pl
from jax.experimental.pallas import tpu as pltpu

_N = 87296
_C = 91
_R = 341
_L = 256
_BN = 7936
_G = 11
_IOU = 0.6
_K = 100


def _prep_kernel(logits_ref, ctr_ref, anc_ref, reg_ref, pack_ref):
    lg = logits_ref[...]
    ct = ctr_ref[...]
    s = jnp.sqrt(jax.nn.sigmoid(lg) * jax.nn.sigmoid(ct))
    mx = jnp.max(s, axis=0, keepdims=True)
    row = lax.broadcasted_iota(jnp.int32, s.shape, 0)
    lbl = jnp.min(jnp.where(s == mx, row, _C), axis=0, keepdims=True)
    pack_ref[...] = jnp.concatenate(
        [anc_ref[...], reg_ref[...], mx, lbl.astype(jnp.float32)], axis=0)


def _nms_kernel(pack_ref, out_ref, x1s, y1s, x2s, y2s, ars, scs):
    ax1 = pack_ref[0]
    ay1 = pack_ref[1]
    ax2 = pack_ref[2]
    ay2 = pack_ref[3]
    cx = 0.5 * (ax1 + ax2)
    cy = 0.5 * (ay1 + ay2)
    w = ax2 - ax1
    h = ay2 - ay1
    x1 = cx - pack_ref[4] * w
    y1 = cy - pack_ref[5] * h
    x2 = cx + pack_ref[6] * w
    y2 = cy + pack_ref[7] * h
    x1s[...] = x1
    y1s[...] = y1
    x2s[...] = x2
    y2s[...] = y2
    ars[...] = jnp.maximum(x2 - x1, 0.0) * jnp.maximum(y2 - y1, 0.0)
    s0 = pack_ref[8]
    scs[...] = s0

    neginf = jnp.float32(-jnp.inf)
    rowio = lax.broadcasted_iota(jnp.int32, (_R, _L), 0)
    colmax0 = jnp.max(s0, axis=0, keepdims=True)
    colrow0 = jnp.min(jnp.where(s0 == colmax0, rowio, _R), axis=0,
                      keepdims=True)

    def body(i, carry):
        colmax, colrow = carry
        lane = lax.broadcasted_iota(jnp.int32, (1, _L), 1)
        m = jnp.max(colmax, axis=1, keepdims=True)
        cand = jnp.where(colmax == m, colrow * _L + lane, _N)
        idx = jnp.min(cand, axis=1, keepdims=True)[0, 0]
        r = lax.shift_right_logical(idx, 8)
        c = lax.bitwise_and(idx, 255)
        sel = lane == c

        def pickf(ref):
            row = ref[pl.ds(r, 1), :]
            return jnp.max(jnp.where(sel, row, neginf), axis=1, keepdims=True)

        bx1 = pickf(x1s)
        by1 = pickf(y1s)
        bx2 = pickf(x2s)
        by2 = pickf(y2s)
        bar = pickf(ars)
        srow = pack_ref[8, pl.ds(r, 1), :]
        bsc = jnp.max(jnp.where(sel, srow, neginf), axis=1, keepdims=True)
        lrow = pack_ref[9, pl.ds(r, 1), :]
        blb = jnp.max(jnp.where(sel, lrow, neginf), axis=1, keepdims=True)

        s = scs[...]
        iw = jnp.maximum(jnp.minimum(x2s[...], bx2)
                         - jnp.maximum(x1s[...], bx1), 0.0)
        ih = jnp.maximum(jnp.minimum(y2s[...], by2)
                         - jnp.maximum(y1s[...], by1), 0.0)
        inter = iw * ih
        iou = inter / (bar + ars[...] - inter)
        new_s = jnp.where(iou > _IOU, neginf, s)
        scs[...] = new_s
        colmax2 = jnp.max(new_s, axis=0, keepdims=True)
        colrow2 = jnp.min(jnp.where(new_s == colmax2, rowio, _R), axis=0,
                          keepdims=True)

        li = lax.broadcasted_iota(jnp.int32, (1, 128), 1)
        v = jnp.where(
            li == 0, bx1,
            jnp.where(li == 1, by1,
                      jnp.where(li == 2, bx2,
                                jnp.where(li == 3, by2,
                                          jnp.where(li == 4, bsc,
                                                    jnp.where(li == 5, blb,
                                                              0.0))))))
        out_ref[pl.ds(i, 1)] = v.reshape(1, 1, 128)
        return (colmax2, colrow2)

    lax.fori_loop(0, _K, body, (colmax0, colrow0), unroll=8)


def _prep_call(class_logits, box_ctrness, anchors, box_regression):
    return pl.pallas_call(
        _prep_kernel,
        grid=(_G,),
        in_specs=[
            pl.BlockSpec((_C, _BN), lambda i: (0, i)),
            pl.BlockSpec((1, _BN), lambda i: (0, i)),
            pl.BlockSpec((4, _BN), lambda i: (0, i)),
            pl.BlockSpec((4, _BN), lambda i: (0, i)),
        ],
        out_specs=pl.BlockSpec((10, _BN), lambda i: (0, i)),
        out_shape=jax.ShapeDtypeStruct((10, _N), jnp.float32),
        compiler_params=pltpu.CompilerParams(
            dimension_semantics=("arbitrary",),
        ),
        name="fcos_prep",
    )(class_logits, box_ctrness, anchors, box_regression)


def _nms_call(pack_p):
    return pl.pallas_call(
        _nms_kernel,
        out_shape=jax.ShapeDtypeStruct((_K, 1, 128), jnp.float32),
        scratch_shapes=[pltpu.VMEM((_R, _L), jnp.float32)] * 6,
        name="fcos_nms",
    )(pack_p)


def kernel(class_logits, box_regression, box_ctrness, anchors):
    pack = _prep_call(class_logits.T, box_ctrness.T,
                      anchors.T, box_regression.T)
    pack_p = pack.reshape(10, _R, _L)
    out = _nms_call(pack_p).reshape(_K, 128)
    pred_boxes = out[:, :4]
    scores = out[:, 4]
    labels = out[:, 5].astype(jnp.int32)
    return pred_boxes, labels, scores
